# Initial kernel scaffold; baseline (speedup 1.0000x reference)
import jax
import jax.numpy as jnp
from jax.experimental import pallas as pl


def kernel(x, kernels, a, b):
    raise NotImplementedError("write your pallas kernel here")


# speedup vs baseline: 2.9959x; 2.9959x over previous
import functools

import jax
import jax.numpy as jnp
from jax.experimental import pallas as pl
from jax.experimental.pallas import tpu as pltpu

NUM_HASHES = 5
TABLE_SIZE = 256
M = 9
R = 4.0
U = 0.99
KH, KW = 3, 3
IN_CH = 96
OUT_CH = 192
H = 224
W = 224
TH = 8
NT = H // TH


def _vote_kernel(x_ref, a_ref, bias_ref, votes_ref):
    acc = jnp.zeros((NUM_HASHES, TH * W), dtype=jnp.float32)
    for dh in range(KH):
        for dw in range(KW):
            xs = x_ref[0, 0, :, dh:dh + TH, dw:dw + W].reshape(IN_CH, TH * W)
            acc += jnp.dot(a_ref[dh, dw], xs, preferred_element_type=jnp.float32)
    q = jnp.floor((acc + bias_ref[:]) / R).astype(jnp.int32)
    v = jnp.abs(jax.lax.rem(q, TABLE_SIZE))
    votes_ref[0] = v.reshape(NUM_HASHES, TH, W)


def _conv_kernel(x_ref, w_ref, mask_ref, out_ref):
    acc = jnp.zeros((OUT_CH, TH * W), dtype=jnp.float32)
    for dh in range(KH):
        for dw in range(KW):
            xs = x_ref[0, 0, :, dh:dh + TH, dw:dw + W].reshape(IN_CH, TH * W)
            acc += jnp.dot(w_ref[dh, dw], xs, preferred_element_type=jnp.float32)
    out_ref[0] = (acc * mask_ref[:]).reshape(OUT_CH, TH, W)


def _build_table(kernels, a, b):
    flat = kernels.reshape(OUT_CH, -1)
    denom = jnp.linalg.norm(flat, axis=1).max()
    flat = U * flat / denom
    norms = jnp.linalg.norm(flat, axis=1, keepdims=True)
    app = jnp.concatenate([norms ** (2 ** (i + 1)) for i in range(M)], axis=1)
    p = jnp.concatenate([flat, app], axis=1)
    h = jnp.floor((p @ a.T + b[None, :]) / R)
    idx = jnp.abs(jax.lax.rem(h.T.astype(jnp.int32), TABLE_SIZE))
    return idx


@jax.jit
def kernel(x, kernels, a, b):
    B = x.shape[0]
    x_pad = jnp.pad(x, ((0, 0), (0, 0), (1, 1), (1, 1)))
    x_tiles = jnp.stack(
        [x_pad[:, :, i * TH:i * TH + TH + 2, :] for i in range(NT)], axis=1)

    ak = a.reshape(NUM_HASHES, IN_CH + 1, KH, KW)
    a_taps = ak[:, :IN_CH].transpose(2, 3, 0, 1)
    bias = (b + 0.5 * ak[:, IN_CH].sum(axis=(1, 2))).reshape(NUM_HASHES, 1)
    votes = pl.pallas_call(
        _vote_kernel,
        grid=(B, NT),
        in_specs=[
            pl.BlockSpec((1, 1, IN_CH, TH + 2, W + 2),
                         lambda bi, i: (bi, i, 0, 0, 0)),
            pl.BlockSpec((KH, KW, NUM_HASHES, IN_CH), lambda bi, i: (0, 0, 0, 0)),
            pl.BlockSpec((NUM_HASHES, 1), lambda bi, i: (0, 0)),
        ],
        out_specs=pl.BlockSpec((1, NUM_HASHES, TH, W),
                               lambda bi, i: (bi, 0, i, 0)),
        out_shape=jax.ShapeDtypeStruct((B, NUM_HASHES, H, W), jnp.int32),
    )(x_tiles, a_taps, bias)

    vflat = votes.transpose(1, 0, 2, 3).reshape(NUM_HASHES, -1)
    counts = (vflat[:, None, :] == jnp.arange(TABLE_SIZE, dtype=jnp.int32)[None, :, None]
              ).sum(axis=2)
    winners = jnp.argmax(counts, axis=1).astype(jnp.int32)
    table = _build_table(kernels, a, b)
    mask = (table[:, :, None] == winners[None, None, :]).any(axis=(0, 2))
    mask = jnp.where(mask.any(), mask, jnp.ones((OUT_CH,), dtype=bool))
    maskf = mask.astype(jnp.float32).reshape(OUT_CH, 1)

    w_taps = kernels.transpose(2, 3, 0, 1)
    out = pl.pallas_call(
        _conv_kernel,
        grid=(B, NT),
        in_specs=[
            pl.BlockSpec((1, 1, IN_CH, TH + 2, W + 2),
                         lambda bi, i: (bi, i, 0, 0, 0)),
            pl.BlockSpec((KH, KW, OUT_CH, IN_CH), lambda bi, i: (0, 0, 0, 0)),
            pl.BlockSpec((OUT_CH, 1), lambda bi, i: (0, 0)),
        ],
        out_specs=pl.BlockSpec((1, OUT_CH, TH, W), lambda bi, i: (bi, 0, i, 0)),
        out_shape=jax.ShapeDtypeStruct((B, OUT_CH, H, W), jnp.float32),
    )(x_tiles, w_taps, maskf)
    return out


# speedup vs baseline: 4.2755x; 1.4271x over previous
import functools

import jax
import jax.numpy as jnp
from jax.experimental import pallas as pl
from jax.experimental.pallas import tpu as pltpu

NUM_HASHES = 5
TABLE_SIZE = 256
M = 9
R = 4.0
U = 0.99
KH, KW = 3, 3
IN_CH = 96
OUT_CH = 192
H = 224
W = 224
TH = 8
NT = H // TH


def _vote_kernel(x_ref, a_ref, bias_ref, votes_ref):
    i = pl.program_id(1)
    xa = x_ref[0, :, pl.ds(i * TH, TH + 8), :]
    acc = jnp.zeros((NUM_HASHES, TH * W), dtype=jnp.float32)
    for dh in range(KH):
        for dw in range(KW):
            xs = xa[:, dh:dh + TH, dw:dw + W]
            acc += jnp.dot(a_ref[dh, dw], xs.reshape(IN_CH, TH * W),
                           preferred_element_type=jnp.float32)
    q = jnp.floor((acc + bias_ref[:]) / R).astype(jnp.int32)
    v = jnp.abs(jax.lax.rem(q, TABLE_SIZE))
    votes_ref[0] = v.reshape(NUM_HASHES, TH, W)


def _conv_kernel(x_ref, w_ref, mask_ref, out_ref):
    i = pl.program_id(1)
    xa = x_ref[0, :, pl.ds(i * TH, TH + 8), :]
    acc = jnp.zeros((OUT_CH, TH * W), dtype=jnp.float32)
    for dh in range(KH):
        for dw in range(KW):
            xs = xa[:, dh:dh + TH, dw:dw + W]
            acc += jnp.dot(w_ref[dh, dw], xs.reshape(IN_CH, TH * W),
                           preferred_element_type=jnp.float32)
    out_ref[0] = (acc * mask_ref[:]).reshape(OUT_CH, TH, W)


def _build_table(kernels, a, b):
    flat = kernels.reshape(OUT_CH, -1)
    denom = jnp.linalg.norm(flat, axis=1).max()
    flat = U * flat / denom
    norms = jnp.linalg.norm(flat, axis=1, keepdims=True)
    app = jnp.concatenate([norms ** (2 ** (i + 1)) for i in range(M)], axis=1)
    p = jnp.concatenate([flat, app], axis=1)
    h = jnp.floor((p @ a.T + b[None, :]) / R)
    idx = jnp.abs(jax.lax.rem(h.T.astype(jnp.int32), TABLE_SIZE))
    return idx


@jax.jit
def kernel(x, kernels, a, b):
    B = x.shape[0]
    x_pad = jnp.pad(x, ((0, 0), (0, 0), (1, 7), (1, 1)))
    x_pad_bf = x_pad.astype(jnp.bfloat16)

    ak = a.reshape(NUM_HASHES, IN_CH + 1, KH, KW)
    a_taps = ak[:, :IN_CH].transpose(2, 3, 0, 1)
    bias = (b + 0.5 * ak[:, IN_CH].sum(axis=(1, 2))).reshape(NUM_HASHES, 1)
    votes = pl.pallas_call(
        _vote_kernel,
        grid=(B, NT),
        in_specs=[
            pl.BlockSpec((1, IN_CH, H + 8, W + 2),
                         lambda bi, i: (bi, 0, 0, 0)),
            pl.BlockSpec((KH, KW, NUM_HASHES, IN_CH), lambda bi, i: (0, 0, 0, 0)),
            pl.BlockSpec((NUM_HASHES, 1), lambda bi, i: (0, 0)),
        ],
        out_specs=pl.BlockSpec((1, NUM_HASHES, TH, W),
                               lambda bi, i: (bi, 0, i, 0)),
        out_shape=jax.ShapeDtypeStruct((B, NUM_HASHES, H, W), jnp.int32),
    )(x_pad, a_taps, bias)

    vflat = votes.transpose(1, 0, 2, 3).reshape(NUM_HASHES, -1)
    counts = (vflat[:, None, :] == jnp.arange(TABLE_SIZE, dtype=jnp.int32)[None, :, None]
              ).sum(axis=2)
    winners = jnp.argmax(counts, axis=1).astype(jnp.int32)
    table = _build_table(kernels, a, b)
    mask = (table[:, :, None] == winners[None, None, :]).any(axis=(0, 2))
    mask = jnp.where(mask.any(), mask, jnp.ones((OUT_CH,), dtype=bool))
    maskf = mask.astype(jnp.float32).reshape(OUT_CH, 1)

    w_taps = kernels.transpose(2, 3, 0, 1).astype(jnp.bfloat16)
    out = pl.pallas_call(
        _conv_kernel,
        grid=(B, NT),
        in_specs=[
            pl.BlockSpec((1, IN_CH, H + 8, W + 2),
                         lambda bi, i: (bi, 0, 0, 0)),
            pl.BlockSpec((KH, KW, OUT_CH, IN_CH), lambda bi, i: (0, 0, 0, 0)),
            pl.BlockSpec((OUT_CH, 1), lambda bi, i: (0, 0)),
        ],
        out_specs=pl.BlockSpec((1, OUT_CH, TH, W), lambda bi, i: (bi, 0, i, 0)),
        out_shape=jax.ShapeDtypeStruct((B, OUT_CH, H, W), jnp.float32),
    )(x_pad_bf, w_taps, maskf)
    return out


# speedup vs baseline: 4.4588x; 1.0429x over previous
import functools

import jax
import jax.numpy as jnp
from jax.experimental import pallas as pl
from jax.experimental.pallas import tpu as pltpu

NUM_HASHES = 5
TABLE_SIZE = 256
M = 9
R = 4.0
U = 0.99
KH, KW = 3, 3
IN_CH = 96
OUT_CH = 192
H = 224
W = 224
TH = 8
NT = H // TH


def _vote_kernel(x_ref, a_ref, bias_ref, counts_ref):
    i = pl.program_id(1)
    xa = x_ref[0, :, pl.ds(i * TH, TH + 8), :]
    acc = jnp.zeros((NUM_HASHES, TH * W), dtype=jnp.float32)
    for dh in range(KH):
        for dw in range(KW):
            xs = xa[:, dh:dh + TH, dw:dw + W]
            acc += jnp.dot(a_ref[dh, dw], xs.reshape(IN_CH, TH * W),
                           preferred_element_type=jnp.float32)
    q = jnp.floor((acc + bias_ref[:]) / R).astype(jnp.int32)
    v = jnp.abs(jax.lax.rem(q, TABLE_SIZE))
    bins = jax.lax.broadcasted_iota(jnp.int32, (TH * W, TABLE_SIZE), 1)
    part = jnp.stack(
        [jnp.sum((v[h][:, None] == bins).astype(jnp.float32), axis=0)
         for h in range(NUM_HASHES)])
    first = jnp.logical_and(pl.program_id(0) == 0, i == 0)

    @pl.when(first)
    def _():
        counts_ref[:] = part

    @pl.when(jnp.logical_not(first))
    def _():
        counts_ref[:] += part


def _conv_kernel(x_ref, w_ref, mask_ref, out_ref):
    i = pl.program_id(1)
    xa = x_ref[0, :, pl.ds(i * TH, TH + 8), :]
    acc = jnp.zeros((OUT_CH, TH * W), dtype=jnp.float32)
    for dh in range(KH):
        for dw in range(KW):
            xs = xa[:, dh:dh + TH, dw:dw + W]
            acc += jnp.dot(w_ref[dh, dw], xs.reshape(IN_CH, TH * W),
                           preferred_element_type=jnp.float32)
    out_ref[0] = (acc * mask_ref[:]).reshape(OUT_CH, TH, W)


def _build_table(kernels, a, b):
    flat = kernels.reshape(OUT_CH, -1)
    denom = jnp.linalg.norm(flat, axis=1).max()
    flat = U * flat / denom
    norms = jnp.linalg.norm(flat, axis=1, keepdims=True)
    app = jnp.concatenate([norms ** (2 ** (i + 1)) for i in range(M)], axis=1)
    p = jnp.concatenate([flat, app], axis=1)
    h = jnp.floor((p @ a.T + b[None, :]) / R)
    idx = jnp.abs(jax.lax.rem(h.T.astype(jnp.int32), TABLE_SIZE))
    return idx


@jax.jit
def kernel(x, kernels, a, b):
    B = x.shape[0]
    x_pad = jnp.pad(x, ((0, 0), (0, 0), (1, 7), (1, 1)))
    x_pad_bf = x_pad.astype(jnp.bfloat16)

    ak = a.reshape(NUM_HASHES, IN_CH + 1, KH, KW)
    a_taps = ak[:, :IN_CH].transpose(2, 3, 0, 1)
    bias = (b + 0.5 * ak[:, IN_CH].sum(axis=(1, 2))).reshape(NUM_HASHES, 1)
    counts = pl.pallas_call(
        _vote_kernel,
        grid=(B, NT),
        in_specs=[
            pl.BlockSpec((1, IN_CH, H + 8, W + 2),
                         lambda bi, i: (bi, 0, 0, 0)),
            pl.BlockSpec((KH, KW, NUM_HASHES, IN_CH), lambda bi, i: (0, 0, 0, 0)),
            pl.BlockSpec((NUM_HASHES, 1), lambda bi, i: (0, 0)),
        ],
        out_specs=pl.BlockSpec((NUM_HASHES, TABLE_SIZE), lambda bi, i: (0, 0)),
        out_shape=jax.ShapeDtypeStruct((NUM_HASHES, TABLE_SIZE), jnp.float32),
    )(x_pad, a_taps, bias)

    winners = jnp.argmax(counts, axis=1).astype(jnp.int32)
    table = _build_table(kernels, a, b)
    mask = (table[:, :, None] == winners[None, None, :]).any(axis=(0, 2))
    mask = jnp.where(mask.any(), mask, jnp.ones((OUT_CH,), dtype=bool))
    maskf = mask.astype(jnp.float32).reshape(OUT_CH, 1)

    w_taps = kernels.transpose(2, 3, 0, 1).astype(jnp.bfloat16)
    out = pl.pallas_call(
        _conv_kernel,
        grid=(B, NT),
        in_specs=[
            pl.BlockSpec((1, IN_CH, H + 8, W + 2),
                         lambda bi, i: (bi, 0, 0, 0)),
            pl.BlockSpec((KH, KW, OUT_CH, IN_CH), lambda bi, i: (0, 0, 0, 0)),
            pl.BlockSpec((OUT_CH, 1), lambda bi, i: (0, 0)),
        ],
        out_specs=pl.BlockSpec((1, OUT_CH, TH, W), lambda bi, i: (bi, 0, i, 0)),
        out_shape=jax.ShapeDtypeStruct((B, OUT_CH, H, W), jnp.float32),
    )(x_pad_bf, w_taps, maskf)
    return out


# speedup vs baseline: 4.6550x; 1.0440x over previous
import functools

import jax
import jax.numpy as jnp
from jax.experimental import pallas as pl
from jax.experimental.pallas import tpu as pltpu

NUM_HASHES = 5
TABLE_SIZE = 256
M = 9
R = 4.0
U = 0.99
KH, KW = 3, 3
IN_CH = 96
OUT_CH = 192
H = 224
W = 224
TH = 16
NT = H // TH


def _vote_kernel(x_ref, a_ref, bias_ref, counts_ref):
    i = pl.program_id(1)
    xa = x_ref[0, :, pl.ds(i * TH, TH + 8), :]
    acc = jnp.zeros((NUM_HASHES, TH * W), dtype=jnp.float32)
    for dh in range(KH):
        for dw in range(KW):
            xs = xa[:, dh:dh + TH, dw:dw + W]
            acc += jnp.dot(a_ref[dh, dw], xs.reshape(IN_CH, TH * W),
                           preferred_element_type=jnp.float32)
    q = jnp.floor((acc + bias_ref[:]) / R).astype(jnp.int32)
    v = jnp.abs(jax.lax.rem(q, TABLE_SIZE))
    bins = jax.lax.broadcasted_iota(jnp.int32, (TH * W, TABLE_SIZE), 1)
    part = jnp.stack(
        [jnp.sum((v[h][:, None] == bins).astype(jnp.float32), axis=0)
         for h in range(NUM_HASHES)])
    first = jnp.logical_and(pl.program_id(0) == 0, i == 0)

    @pl.when(first)
    def _():
        counts_ref[:] = part

    @pl.when(jnp.logical_not(first))
    def _():
        counts_ref[:] += part


def _conv_kernel(x_ref, w_ref, mask_ref, out_ref):
    i = pl.program_id(1)
    xa = x_ref[0, :, pl.ds(i * TH, TH + 8), :]
    acc = jnp.zeros((OUT_CH, TH * W), dtype=jnp.float32)
    for dh in range(KH):
        for dw in range(KW):
            xs = xa[:, dh:dh + TH, dw:dw + W]
            acc += jnp.dot(w_ref[dh, dw], xs.reshape(IN_CH, TH * W),
                           preferred_element_type=jnp.float32)
    out_ref[0] = (acc * mask_ref[:]).reshape(OUT_CH, TH, W)


def _build_table(kernels, a, b):
    flat = kernels.reshape(OUT_CH, -1)
    denom = jnp.linalg.norm(flat, axis=1).max()
    flat = U * flat / denom
    norms = jnp.linalg.norm(flat, axis=1, keepdims=True)
    app = jnp.concatenate([norms ** (2 ** (i + 1)) for i in range(M)], axis=1)
    p = jnp.concatenate([flat, app], axis=1)
    h = jnp.floor((p @ a.T + b[None, :]) / R)
    idx = jnp.abs(jax.lax.rem(h.T.astype(jnp.int32), TABLE_SIZE))
    return idx


@jax.jit
def kernel(x, kernels, a, b):
    B = x.shape[0]
    x_pad = jnp.pad(x, ((0, 0), (0, 0), (1, 7), (1, 1)))
    x_pad_bf = x_pad.astype(jnp.bfloat16)

    ak = a.reshape(NUM_HASHES, IN_CH + 1, KH, KW)
    a_taps = ak[:, :IN_CH].transpose(2, 3, 0, 1)
    bias = (b + 0.5 * ak[:, IN_CH].sum(axis=(1, 2))).reshape(NUM_HASHES, 1)
    counts = pl.pallas_call(
        _vote_kernel,
        grid=(B, NT),
        in_specs=[
            pl.BlockSpec((1, IN_CH, H + 8, W + 2),
                         lambda bi, i: (bi, 0, 0, 0)),
            pl.BlockSpec((KH, KW, NUM_HASHES, IN_CH), lambda bi, i: (0, 0, 0, 0)),
            pl.BlockSpec((NUM_HASHES, 1), lambda bi, i: (0, 0)),
        ],
        out_specs=pl.BlockSpec((NUM_HASHES, TABLE_SIZE), lambda bi, i: (0, 0)),
        out_shape=jax.ShapeDtypeStruct((NUM_HASHES, TABLE_SIZE), jnp.float32),
    )(x_pad, a_taps, bias)

    winners = jnp.argmax(counts, axis=1).astype(jnp.int32)
    table = _build_table(kernels, a, b)
    mask = (table[:, :, None] == winners[None, None, :]).any(axis=(0, 2))
    mask = jnp.where(mask.any(), mask, jnp.ones((OUT_CH,), dtype=bool))
    maskf = mask.astype(jnp.float32).reshape(OUT_CH, 1)

    w_taps = kernels.transpose(2, 3, 0, 1).astype(jnp.bfloat16)
    out = pl.pallas_call(
        _conv_kernel,
        grid=(B, NT),
        in_specs=[
            pl.BlockSpec((1, IN_CH, H + 8, W + 2),
                         lambda bi, i: (bi, 0, 0, 0)),
            pl.BlockSpec((KH, KW, OUT_CH, IN_CH), lambda bi, i: (0, 0, 0, 0)),
            pl.BlockSpec((OUT_CH, 1), lambda bi, i: (0, 0)),
        ],
        out_specs=pl.BlockSpec((1, OUT_CH, TH, W), lambda bi, i: (bi, 0, i, 0)),
        out_shape=jax.ShapeDtypeStruct((B, OUT_CH, H, W), jnp.float32),
    )(x_pad_bf, w_taps, maskf)
    return out


# speedup vs baseline: 5.2657x; 1.1312x over previous
import functools

import jax
import jax.numpy as jnp
from jax import lax
from jax.experimental import pallas as pl
from jax.experimental.pallas import tpu as pltpu
from jax.experimental.pallas import tpu_sc as plsc

NUM_HASHES = 5
TABLE_SIZE = 256
M = 9
R = 4.0
U = 0.99
KH, KW = 3, 3
IN_CH = 96
OUT_CH = 192
H = 224
W = 224
TH = 16
NT = H // TH


def _vote_kernel(x_ref, a_ref, bias_ref, counts_ref):
    i = pl.program_id(1)
    xa = x_ref[0, :, pl.ds(i * TH, TH + 8), :]
    acc = jnp.zeros((NUM_HASHES, TH * W), dtype=jnp.float32)
    for dh in range(KH):
        for dw in range(KW):
            xs = xa[:, dh:dh + TH, dw:dw + W]
            acc += jnp.dot(a_ref[dh, dw], xs.reshape(IN_CH, TH * W),
                           preferred_element_type=jnp.float32)
    q = jnp.floor((acc + bias_ref[:]) / R).astype(jnp.int32)
    v = jnp.abs(jax.lax.rem(q, TABLE_SIZE))
    counts_ref[0] = v.reshape(NUM_HASHES, TH, W)


def _conv_kernel(x_ref, w_ref, mask_ref, out_ref):
    i = pl.program_id(1)
    xa = x_ref[0, :, pl.ds(i * TH, TH + 8), :]
    acc = jnp.zeros((OUT_CH, TH * W), dtype=jnp.float32)
    for dh in range(KH):
        for dw in range(KW):
            xs = xa[:, dh:dh + TH, dw:dw + W]
            acc += jnp.dot(w_ref[dh, dw], xs.reshape(IN_CH, TH * W),
                           preferred_element_type=jnp.float32)
    out_ref[0] = (acc * mask_ref[:]).reshape(OUT_CH, TH, W)


SC_NC = 2
SC_NS = 16
SC_NW = SC_NC * SC_NS
NVOTES = 2 * NUM_HASHES * H * W
VROWS = 2 * NUM_HASHES
VCOLS = H * W
CHUNK = VCOLS // SC_NW
NBINS = NUM_HASHES * TABLE_SIZE


def _sc_hist_kernel(votes_hbm, out_hbm, buf, counts):
    wid = lax.axis_index("s") * SC_NC + lax.axis_index("c")
    base = wid * CHUNK

    @pl.loop(0, NBINS // 16)
    def _zero(k):
        counts[pl.ds(k * 16, 16)] = jnp.zeros((16,), jnp.int32)

    for r in range(VROWS):
        pltpu.sync_copy(votes_hbm.at[pl.ds(r * VCOLS + base, CHUNK)], buf)
        off = (r % NUM_HASHES) * TABLE_SIZE

        @pl.loop(0, CHUNK // 16)
        def _acc(k):
            v = buf[pl.ds(k * 16, 16)]
            plsc.addupdate_scatter(counts, [v + off],
                                   jnp.ones((16,), jnp.int32))

    pltpu.sync_copy(counts, out_hbm.at[wid])


_sc_hist = functools.partial(
    pl.kernel,
    out_type=jax.ShapeDtypeStruct((SC_NW, NBINS), jnp.int32),
    mesh=plsc.VectorSubcoreMesh(core_axis_name="c", subcore_axis_name="s",
                                num_cores=SC_NC, num_subcores=SC_NS),
    scratch_types=[
        pltpu.VMEM((CHUNK,), jnp.int32),
        pltpu.VMEM((NBINS,), jnp.int32),
    ],
    compiler_params=pltpu.CompilerParams(needs_layout_passes=False),
)(_sc_hist_kernel)


def _build_table(kernels, a, b):
    flat = kernels.reshape(OUT_CH, -1)
    denom = jnp.linalg.norm(flat, axis=1).max()
    flat = U * flat / denom
    norms = jnp.linalg.norm(flat, axis=1, keepdims=True)
    app = jnp.concatenate([norms ** (2 ** (i + 1)) for i in range(M)], axis=1)
    p = jnp.concatenate([flat, app], axis=1)
    h = jnp.floor((p @ a.T + b[None, :]) / R)
    idx = jnp.abs(jax.lax.rem(h.T.astype(jnp.int32), TABLE_SIZE))
    return idx


@jax.jit
def kernel(x, kernels, a, b):
    B = x.shape[0]
    x_pad = jnp.pad(x, ((0, 0), (0, 0), (1, 7), (1, 1)))
    x_pad_bf = x_pad.astype(jnp.bfloat16)

    ak = a.reshape(NUM_HASHES, IN_CH + 1, KH, KW)
    a_taps = ak[:, :IN_CH].transpose(2, 3, 0, 1)
    bias = (b + 0.5 * ak[:, IN_CH].sum(axis=(1, 2))).reshape(NUM_HASHES, 1)
    votes = pl.pallas_call(
        _vote_kernel,
        grid=(B, NT),
        in_specs=[
            pl.BlockSpec((1, IN_CH, H + 8, W + 2),
                         lambda bi, i: (bi, 0, 0, 0)),
            pl.BlockSpec((KH, KW, NUM_HASHES, IN_CH), lambda bi, i: (0, 0, 0, 0)),
            pl.BlockSpec((NUM_HASHES, 1), lambda bi, i: (0, 0)),
        ],
        out_specs=pl.BlockSpec((1, NUM_HASHES, TH, W),
                               lambda bi, i: (bi, 0, i, 0)),
        out_shape=jax.ShapeDtypeStruct((B, NUM_HASHES, H, W), jnp.int32),
    )(x_pad, a_taps, bias)

    partials = _sc_hist(votes.reshape(NVOTES))
    counts = partials.sum(axis=0).reshape(NUM_HASHES, TABLE_SIZE)

    winners = jnp.argmax(counts, axis=1).astype(jnp.int32)
    table = _build_table(kernels, a, b)
    mask = (table[:, :, None] == winners[None, None, :]).any(axis=(0, 2))
    mask = jnp.where(mask.any(), mask, jnp.ones((OUT_CH,), dtype=bool))
    maskf = mask.astype(jnp.float32).reshape(OUT_CH, 1)

    w_taps = kernels.transpose(2, 3, 0, 1).astype(jnp.bfloat16)
    out = pl.pallas_call(
        _conv_kernel,
        grid=(B, NT),
        in_specs=[
            pl.BlockSpec((1, IN_CH, H + 8, W + 2),
                         lambda bi, i: (bi, 0, 0, 0)),
            pl.BlockSpec((KH, KW, OUT_CH, IN_CH), lambda bi, i: (0, 0, 0, 0)),
            pl.BlockSpec((OUT_CH, 1), lambda bi, i: (0, 0)),
        ],
        out_specs=pl.BlockSpec((1, OUT_CH, TH, W), lambda bi, i: (bi, 0, i, 0)),
        out_shape=jax.ShapeDtypeStruct((B, OUT_CH, H, W), jnp.float32),
    )(x_pad_bf, w_taps, maskf)
    return out


# speedup vs baseline: 5.3617x; 1.0182x over previous
import functools

import jax
import jax.numpy as jnp
from jax import lax
from jax.experimental import pallas as pl
from jax.experimental.pallas import tpu as pltpu
from jax.experimental.pallas import tpu_sc as plsc

NUM_HASHES = 5
TABLE_SIZE = 256
M = 9
R = 4.0
U = 0.99
KH, KW = 3, 3
IN_CH = 96
OUT_CH = 192
H = 224
W = 224
TH = 16
NT = H // TH


def _vote_kernel(x_ref, a_ref, bias_ref, counts_ref):
    i = pl.program_id(1)
    xa = x_ref[0, :, pl.ds(i * TH, TH + 8), :]
    acc = jnp.zeros((NUM_HASHES, TH * W), dtype=jnp.float32)
    for dh in range(KH):
        for dw in range(KW):
            xs = xa[:, dh:dh + TH, dw:dw + W]
            acc += jnp.dot(a_ref[dh, dw], xs.reshape(IN_CH, TH * W),
                           preferred_element_type=jnp.float32)
    q = jnp.floor((acc + bias_ref[:]) / R).astype(jnp.int32)
    v = jnp.abs(jax.lax.rem(q, TABLE_SIZE))
    counts_ref[0] = v.reshape(NUM_HASHES, TH, W)


def _conv_kernel(x_ref, w_ref, mask_ref, out_ref):
    i = pl.program_id(1)
    xa = x_ref[0, :, pl.ds(i * TH, TH + 8), :].astype(jnp.bfloat16)
    acc = jnp.zeros((OUT_CH, TH * W), dtype=jnp.float32)
    for dh in range(KH):
        for dw in range(KW):
            xs = xa[:, dh:dh + TH, dw:dw + W]
            acc += jnp.dot(w_ref[dh, dw], xs.reshape(IN_CH, TH * W),
                           preferred_element_type=jnp.float32)
    out_ref[0] = (acc * mask_ref[:]).reshape(OUT_CH, TH, W)


SC_NC = 2
SC_NS = 16
SC_NW = SC_NC * SC_NS
NVOTES = 2 * NUM_HASHES * H * W
VROWS = 2 * NUM_HASHES
VCOLS = H * W
CHUNK = VCOLS // SC_NW
NBINS = NUM_HASHES * TABLE_SIZE


def _sc_hist_kernel(votes_hbm, out_hbm, buf, counts):
    wid = lax.axis_index("s") * SC_NC + lax.axis_index("c")
    base = wid * CHUNK

    @pl.loop(0, NBINS // 16)
    def _zero(k):
        counts[pl.ds(k * 16, 16)] = jnp.zeros((16,), jnp.int32)

    for r in range(VROWS):
        pltpu.sync_copy(votes_hbm.at[pl.ds(r * VCOLS + base, CHUNK)], buf)
        off = (r % NUM_HASHES) * TABLE_SIZE

        @pl.loop(0, CHUNK // 16)
        def _acc(k):
            v = buf[pl.ds(k * 16, 16)]
            plsc.addupdate_scatter(counts, [v + off],
                                   jnp.ones((16,), jnp.int32))

    pltpu.sync_copy(counts, out_hbm.at[wid])


_sc_hist = functools.partial(
    pl.kernel,
    out_type=jax.ShapeDtypeStruct((SC_NW, NBINS), jnp.int32),
    mesh=plsc.VectorSubcoreMesh(core_axis_name="c", subcore_axis_name="s",
                                num_cores=SC_NC, num_subcores=SC_NS),
    scratch_types=[
        pltpu.VMEM((CHUNK,), jnp.int32),
        pltpu.VMEM((NBINS,), jnp.int32),
    ],
    compiler_params=pltpu.CompilerParams(needs_layout_passes=False),
)(_sc_hist_kernel)


def _build_table(kernels, a, b):
    flat = kernels.reshape(OUT_CH, -1)
    denom = jnp.linalg.norm(flat, axis=1).max()
    flat = U * flat / denom
    norms = jnp.linalg.norm(flat, axis=1, keepdims=True)
    app = jnp.concatenate([norms ** (2 ** (i + 1)) for i in range(M)], axis=1)
    p = jnp.concatenate([flat, app], axis=1)
    h = jnp.floor((p @ a.T + b[None, :]) / R)
    idx = jnp.abs(jax.lax.rem(h.T.astype(jnp.int32), TABLE_SIZE))
    return idx


@jax.jit
def kernel(x, kernels, a, b):
    B = x.shape[0]
    x_pad = jnp.pad(x, ((0, 0), (0, 0), (1, 7), (1, 1)))

    ak = a.reshape(NUM_HASHES, IN_CH + 1, KH, KW)
    a_taps = ak[:, :IN_CH].transpose(2, 3, 0, 1)
    bias = (b + 0.5 * ak[:, IN_CH].sum(axis=(1, 2))).reshape(NUM_HASHES, 1)
    votes = pl.pallas_call(
        _vote_kernel,
        grid=(B, NT),
        in_specs=[
            pl.BlockSpec((1, IN_CH, H + 8, W + 2),
                         lambda bi, i: (bi, 0, 0, 0)),
            pl.BlockSpec((KH, KW, NUM_HASHES, IN_CH), lambda bi, i: (0, 0, 0, 0)),
            pl.BlockSpec((NUM_HASHES, 1), lambda bi, i: (0, 0)),
        ],
        out_specs=pl.BlockSpec((1, NUM_HASHES, TH, W),
                               lambda bi, i: (bi, 0, i, 0)),
        out_shape=jax.ShapeDtypeStruct((B, NUM_HASHES, H, W), jnp.int32),
    )(x_pad, a_taps, bias)

    partials = _sc_hist(votes.reshape(NVOTES))
    counts = partials.sum(axis=0).reshape(NUM_HASHES, TABLE_SIZE)

    winners = jnp.argmax(counts, axis=1).astype(jnp.int32)
    table = _build_table(kernels, a, b)
    mask = (table[:, :, None] == winners[None, None, :]).any(axis=(0, 2))
    mask = jnp.where(mask.any(), mask, jnp.ones((OUT_CH,), dtype=bool))
    maskf = mask.astype(jnp.float32).reshape(OUT_CH, 1)

    w_taps = kernels.transpose(2, 3, 0, 1).astype(jnp.bfloat16)
    out = pl.pallas_call(
        _conv_kernel,
        grid=(B, NT),
        in_specs=[
            pl.BlockSpec((1, IN_CH, H + 8, W + 2),
                         lambda bi, i: (bi, 0, 0, 0)),
            pl.BlockSpec((KH, KW, OUT_CH, IN_CH), lambda bi, i: (0, 0, 0, 0)),
            pl.BlockSpec((OUT_CH, 1), lambda bi, i: (0, 0)),
        ],
        out_specs=pl.BlockSpec((1, OUT_CH, TH, W), lambda bi, i: (bi, 0, i, 0)),
        out_shape=jax.ShapeDtypeStruct((B, OUT_CH, H, W), jnp.float32),
    )(x_pad, w_taps, maskf)
    return out
